# baseline (speedup 1.0000x reference)
import functools

import jax
import jax.numpy as jnp
from jax import lax
from jax.experimental import pallas as pl
from jax.experimental.pallas import tpu as pltpu
from jax.experimental.pallas import tpu_sc as plsc

EXPERTS = 64
TOPK = 8
TOKENS = 32768
DMODEL = 768
NC, NS = 2, 16
NW = NC * NS
TOK_PER_W = TOKENS // NW
GROUPS = TOK_PER_W // 16
MM_TILE = 4096
OUT_ROWS = TOKENS * TOPK // 128
W_ROWS = TOK_PER_W * TOPK // 128


def _logits_body(w_ref, h_ref, out_ref):
    acc = lax.dot_general(
        h_ref[...], w_ref[...], (((1,), (1,)), ((), ())),
        preferred_element_type=jnp.float32)
    out_ref[...] = acc.T


def _logits_t(hf, W):
    return pl.pallas_call(
        _logits_body,
        grid=(TOKENS // MM_TILE,),
        in_specs=[
            pl.BlockSpec((EXPERTS, DMODEL), lambda i: (0, 0)),
            pl.BlockSpec((MM_TILE, DMODEL), lambda i: (i, 0)),
        ],
        out_specs=pl.BlockSpec((EXPERTS, MM_TILE), lambda i: (0, i)),
        out_shape=jax.ShapeDtypeStruct((EXPERTS, TOKENS), jnp.float32),
    )(W, hf)


def _topk_tec(lgT, ids_out, w_out, lg_v, ids_v, w_v):
    c = lax.axis_index("c")
    s = lax.axis_index("s")
    wid = s * NC + c
    base = wid * TOK_PER_W
    pltpu.sync_copy(lgT.at[:, pl.ds(base, TOK_PER_W)], lg_v)
    lanes = lax.iota(jnp.int32, 16)
    lane8 = lanes * TOPK
    one = jnp.full((16,), 1, jnp.int32)

    def topk_group(g16):
        def expert(e):
            return lg_v[e, pl.ds(g16, 16)], one * e

        vs = [None] * TOPK
        ix = [None] * TOPK
        vs[0], ix[0] = expert(0)
        for e in range(1, TOPK):
            x, xi = expert(e)
            for j in range(e):
                cnd = x > vs[j]
                vs[j], x = jnp.where(cnd, x, vs[j]), jnp.where(cnd, vs[j], x)
                ix[j], xi = jnp.where(cnd, xi, ix[j]), jnp.where(cnd, ix[j], xi)
            vs[e], ix[e] = x, xi
        for e in range(TOPK, EXPERTS):
            x, xi = expert(e)
            for j in range(TOPK):
                cnd = x > vs[j]
                vs[j], x = jnp.where(cnd, x, vs[j]), jnp.where(cnd, vs[j], x)
                ix[j], xi = jnp.where(cnd, xi, ix[j]), jnp.where(cnd, ix[j], xi)
        es = [jnp.exp(t - vs[0]) for t in vs]
        tot = es[0]
        for t in es[1:]:
            tot = tot + t
        for j in range(TOPK):
            ids_v[j, pl.ds(g16, 16)] = ix[j]
            w_v[j, pl.ds(g16, 16)] = es[j] / tot

    def group(g, _):
        topk_group(g * 32)
        topk_group(g * 32 + 16)
        return 0

    lax.fori_loop(0, GROUPS // 2, group, 0)
    pltpu.sync_copy(ids_v, ids_out.at[:, pl.ds(base, TOK_PER_W)])
    pltpu.sync_copy(w_v, w_out.at[:, pl.ds(base, TOK_PER_W)])


_topk_call = pl.kernel(
    _topk_tec,
    out_type=[
        jax.ShapeDtypeStruct((TOPK, TOKENS), jnp.int32),
        jax.ShapeDtypeStruct((TOPK, TOKENS), jnp.float32),
    ],
    mesh=plsc.VectorSubcoreMesh(
        core_axis_name="c", subcore_axis_name="s",
        num_cores=NC, num_subcores=NS),
    compiler_params=pltpu.CompilerParams(needs_layout_passes=False),
    scratch_types=[
        pltpu.VMEM((EXPERTS, TOK_PER_W), jnp.float32),
        pltpu.VMEM((TOPK, TOK_PER_W), jnp.int32),
        pltpu.VMEM((TOPK, TOK_PER_W), jnp.float32),
    ],
)


def kernel(h, W):
    hf = h.reshape(TOKENS, DMODEL)
    lgT = _logits_t(hf, W)
    ids_sm, w_sm = _topk_call(lgT)
    return (ids_sm.T, w_sm.T, jnp.float32(0.0))


# speedup vs baseline: 1.4843x; 1.4843x over previous
import jax
import jax.numpy as jnp
from jax import lax
from jax.experimental import pallas as pl
from jax.experimental.pallas import tpu as pltpu
from jax.experimental.pallas import tpu_sc as plsc

EXPERTS = 64
TOPK = 8
TOKENS = 32768
DMODEL = 768
NC, NS = 2, 16
NW = NC * NS
TOK_PER_W = TOKENS // NW
GROUPS = TOK_PER_W // 16
MM_TILE = 4096


def _logits_body(w_ref, h_ref, out_ref):
    acc = lax.dot_general(
        h_ref[...], w_ref[...], (((1,), (1,)), ((), ())),
        preferred_element_type=jnp.float32)
    out_ref[...] = acc.T


def _logits_t(hf, W):
    return pl.pallas_call(
        _logits_body,
        grid=(TOKENS // MM_TILE,),
        in_specs=[
            pl.BlockSpec((EXPERTS, DMODEL), lambda i: (0, 0)),
            pl.BlockSpec((MM_TILE, DMODEL), lambda i: (i, 0)),
        ],
        out_specs=pl.BlockSpec((EXPERTS, MM_TILE), lambda i: (0, i)),
        out_shape=jax.ShapeDtypeStruct((EXPERTS, TOKENS), jnp.float32),
    )(W, hf)


def _topk_tec(lgT, ids_out, w_out, lg_v, ids_v, w_v):
    c = lax.axis_index("c")
    s = lax.axis_index("s")
    wid = s * NC + c
    base = wid * TOK_PER_W
    pltpu.sync_copy(lgT.at[:, pl.ds(base, TOK_PER_W)], lg_v)
    one = jnp.full((16,), 1, jnp.int32)

    def group(g, _):
        g16 = g * 16

        def expert(e):
            return lg_v[e, pl.ds(g16, 16)], one * e

        vs = [None] * TOPK
        ix = [None] * TOPK
        vs[0], ix[0] = expert(0)
        for e in range(1, TOPK):
            x, xi = expert(e)
            for j in range(e):
                cnd = x > vs[j]
                vs[j], x = jnp.where(cnd, x, vs[j]), jnp.where(cnd, vs[j], x)
                ix[j], xi = jnp.where(cnd, xi, ix[j]), jnp.where(cnd, ix[j], xi)
            vs[e], ix[e] = x, xi
        for e in range(TOPK, EXPERTS):
            x, xi = expert(e)
            for j in range(TOPK):
                cnd = x > vs[j]
                vs[j], x = jnp.where(cnd, x, vs[j]), jnp.where(cnd, vs[j], x)
                ix[j], xi = jnp.where(cnd, xi, ix[j]), jnp.where(cnd, ix[j], xi)
        es = [jnp.exp(t - vs[0]) for t in vs]
        tot = es[0]
        for t in es[1:]:
            tot = tot + t
        for j in range(TOPK):
            ids_v[j, pl.ds(g16, 16)] = ix[j]
            w_v[j, pl.ds(g16, 16)] = es[j] / tot
        return 0

    lax.fori_loop(0, GROUPS, group, 0)
    pltpu.sync_copy(ids_v, ids_out.at[:, pl.ds(base, TOK_PER_W)])
    pltpu.sync_copy(w_v, w_out.at[:, pl.ds(base, TOK_PER_W)])


_topk_call = pl.kernel(
    _topk_tec,
    out_type=[
        jax.ShapeDtypeStruct((TOPK, TOKENS), jnp.int32),
        jax.ShapeDtypeStruct((TOPK, TOKENS), jnp.float32),
    ],
    mesh=plsc.VectorSubcoreMesh(
        core_axis_name="c", subcore_axis_name="s",
        num_cores=NC, num_subcores=NS),
    compiler_params=pltpu.CompilerParams(needs_layout_passes=False),
    scratch_types=[
        pltpu.VMEM((EXPERTS, TOK_PER_W), jnp.float32),
        pltpu.VMEM((TOPK, TOK_PER_W), jnp.int32),
        pltpu.VMEM((TOPK, TOK_PER_W), jnp.float32),
    ],
)


def kernel(h, W):
    hf = h.reshape(TOKENS, DMODEL)
    lgT = _logits_t(hf, W)
    ids_sm, w_sm = _topk_call(lgT)
    return (ids_sm.T, w_sm.T, jnp.float32(0.0))
